# baseline (speedup 1.0000x reference)
import functools

import jax
import jax.numpy as jnp
from jax import lax
from jax.experimental import pallas as pl
from jax.experimental.pallas import tpu as pltpu
from jax.experimental.pallas import tpu_sc as plsc

ALPHA = 0.5
_F = 1.0 - ALPHA

_NC = 2
_NS = 16
_NW = _NC * _NS


def _make_sc_gather(n_rows, table_rows, dim):
    rows_per_w = n_rows // _NW
    mesh = plsc.VectorSubcoreMesh(core_axis_name="c", subcore_axis_name="s")

    @functools.partial(
        pl.kernel,
        mesh=mesh,
        compiler_params=pltpu.CompilerParams(use_tc_tiling_on_sc=False),
        out_type=jax.ShapeDtypeStruct((n_rows, dim), jnp.float32),
        scratch_types=[
            pltpu.VMEM((rows_per_w,), jnp.int32),
            pltpu.VMEM((rows_per_w, dim), jnp.float32),
            pltpu.SemaphoreType.DMA,
        ],
    )
    def gather_rows(table_hbm, idx_hbm, out_hbm, idx_v, rows_v, sem):
        wid = lax.axis_index("s") * _NC + lax.axis_index("c")
        base = wid * rows_per_w
        pltpu.sync_copy(idx_hbm.at[pl.ds(base, rows_per_w)], idx_v)
        pltpu.async_copy(table_hbm.at[idx_v], rows_v, sem).wait()
        pltpu.sync_copy(rows_v, out_hbm.at[pl.ds(base, rows_per_w)])

    return gather_rows


def _matmul_body(e_ref, w_ref, b_ref, out_ref):
    out_ref[...] = (
        jnp.dot(e_ref[...], w_ref[...], preferred_element_type=jnp.float32)
        + b_ref[...]
    )


def _loss_body(e_ref, lc_ref, lr_ref, cb_ref, out_ref):
    e = e_ref[...]
    m = (lc_ref[...] == lr_ref[...]).astype(jnp.float32)
    s = jnp.dot(m, e, preferred_element_type=jnp.float32)
    cnt = jnp.sum(m, axis=1, keepdims=True)
    cb = cb_ref[...]
    cbn = cb - _F * (cnt * cb - s)
    r = e - cbn
    out_ref[0, 0] = jnp.sum(r * r) / (e.shape[0] * e.shape[1])


def kernel(embedding, labels, centers, W, b):
    B, D = embedding.shape
    U = W.shape[1]


    NB = 2048
    logits = pl.pallas_call(
        _matmul_body,
        grid=(pl.cdiv(U, NB),),
        in_specs=[
            pl.BlockSpec((B, D), lambda i: (0, 0)),
            pl.BlockSpec((D, NB), lambda i: (0, i)),
            pl.BlockSpec((1, NB), lambda i: (0, i)),
        ],
        out_specs=pl.BlockSpec((B, NB), lambda i: (0, i)),
        out_shape=jax.ShapeDtypeStruct((B, U), jnp.float32),
        compiler_params=pltpu.CompilerParams(
            dimension_semantics=("parallel",)
        ),
    )(embedding, W, b.reshape(1, U))


    return (logits, jnp.float32(0.0))


# speedup vs baseline: 2.2083x; 2.2083x over previous
import functools

import jax
import jax.numpy as jnp
from jax import lax
from jax.experimental import pallas as pl
from jax.experimental.pallas import tpu as pltpu
from jax.experimental.pallas import tpu_sc as plsc

ALPHA = 0.5
_F = 1.0 - ALPHA

_NC = 2
_NS = 16
_NW = _NC * _NS


def _make_sc_gather(n_rows, table_rows, dim):
    rows_per_w = n_rows // _NW
    mesh = plsc.VectorSubcoreMesh(core_axis_name="c", subcore_axis_name="s")

    @functools.partial(
        pl.kernel,
        mesh=mesh,
        compiler_params=pltpu.CompilerParams(use_tc_tiling_on_sc=False),
        out_type=jax.ShapeDtypeStruct((n_rows, dim), jnp.float32),
        scratch_types=[
            pltpu.VMEM((rows_per_w,), jnp.int32),
            pltpu.VMEM((rows_per_w, dim), jnp.float32),
            pltpu.SemaphoreType.DMA,
        ],
    )
    def gather_rows(table_hbm, idx_hbm, out_hbm, idx_v, rows_v, sem):
        wid = lax.axis_index("s") * _NC + lax.axis_index("c")
        base = wid * rows_per_w
        pltpu.sync_copy(idx_hbm.at[pl.ds(base, rows_per_w)], idx_v)
        pltpu.async_copy(table_hbm.at[idx_v], rows_v, sem).wait()
        pltpu.sync_copy(rows_v, out_hbm.at[pl.ds(base, rows_per_w)])

    return gather_rows


def _matmul_t_body(w_ref, et_ref, b_ref, out_ref):
    wt_et = lax.dot_general(
        w_ref[...], et_ref[...],
        (((0,), (0,)), ((), ())),
        preferred_element_type=jnp.float32,
    )
    ones = jnp.ones((1, out_ref.shape[1]), dtype=jnp.float32)
    bias = lax.dot_general(
        b_ref[...], ones,
        (((0,), (0,)), ((), ())),
        preferred_element_type=jnp.float32,
    )
    out_ref[...] = wt_et + bias


def _loss_body(e_ref, lc_ref, lr_ref, cb_ref, out_ref):
    e = e_ref[...]
    m = (lc_ref[...] == lr_ref[...]).astype(jnp.float32)
    s = jnp.dot(m, e, preferred_element_type=jnp.float32)
    cnt = jnp.sum(m, axis=1, keepdims=True)
    cb = cb_ref[...]
    cbn = cb - _F * (cnt * cb - s)
    r = e - cbn
    out_ref[0, 0] = jnp.sum(r * r) / (e.shape[0] * e.shape[1])


def kernel(embedding, labels, centers, W, b):
    B, D = embedding.shape
    U = W.shape[1]

    cb = _make_sc_gather(B, centers.shape[0], D)(centers, labels)

    NB = 2048
    et = embedding.T
    logits_t = pl.pallas_call(
        _matmul_t_body,
        grid=(pl.cdiv(U, NB),),
        in_specs=[
            pl.BlockSpec((D, NB), lambda i: (0, i)),
            pl.BlockSpec((D, B), lambda i: (0, 0)),
            pl.BlockSpec((1, NB), lambda i: (0, i)),
        ],
        out_specs=pl.BlockSpec((NB, B), lambda i: (i, 0)),
        out_shape=jax.ShapeDtypeStruct((U, B), jnp.float32),
        compiler_params=pltpu.CompilerParams(
            dimension_semantics=("parallel",)
        ),
    )(W, et, b.reshape(1, U))

    loss = pl.pallas_call(
        _loss_body,
        out_specs=pl.BlockSpec(memory_space=pltpu.SMEM),
        out_shape=jax.ShapeDtypeStruct((1, 1), jnp.float32),
    )(embedding, labels.reshape(B, 1), labels.reshape(1, B), cb)

    return (jnp.transpose(logits_t), loss[0, 0])


# speedup vs baseline: 2.3097x; 1.0459x over previous
import functools

import jax
import jax.numpy as jnp
from jax import lax
from jax.experimental import pallas as pl
from jax.experimental.pallas import tpu as pltpu
from jax.experimental.pallas import tpu_sc as plsc

ALPHA = 0.5
_F = 1.0 - ALPHA

_NC = 2
_NS = 16
_NW = _NC * _NS


def _make_sc_gather(n_rows, table_rows, dim):
    rows_per_w = n_rows // _NW
    mesh = plsc.VectorSubcoreMesh(core_axis_name="c", subcore_axis_name="s")

    @functools.partial(
        pl.kernel,
        mesh=mesh,
        compiler_params=pltpu.CompilerParams(use_tc_tiling_on_sc=False),
        out_type=jax.ShapeDtypeStruct((n_rows, dim), jnp.float32),
        scratch_types=[
            pltpu.VMEM((rows_per_w,), jnp.int32),
            pltpu.VMEM((rows_per_w, dim), jnp.float32),
            pltpu.SemaphoreType.DMA,
        ],
    )
    def gather_rows(table_hbm, idx_hbm, out_hbm, idx_v, rows_v, sem):
        wid = lax.axis_index("s") * _NC + lax.axis_index("c")
        base = wid * rows_per_w
        pltpu.sync_copy(idx_hbm.at[pl.ds(base, rows_per_w)], idx_v)
        pltpu.async_copy(table_hbm.at[idx_v], rows_v, sem).wait()
        pltpu.sync_copy(rows_v, out_hbm.at[pl.ds(base, rows_per_w)])

    return gather_rows


def _matmul_t_body(w_ref, et_ref, b_ref, out_ref):
    wt_et = lax.dot_general(
        w_ref[...], et_ref[...],
        (((0,), (0,)), ((), ())),
        preferred_element_type=jnp.float32,
    )
    ones = jnp.ones((1, out_ref.shape[1]), dtype=jnp.float32)
    bias = lax.dot_general(
        b_ref[...].reshape(1, b_ref.shape[0]), ones,
        (((0,), (0,)), ((), ())),
        preferred_element_type=jnp.float32,
    )
    out_ref[...] = wt_et + bias


def _loss_body(e_ref, lc_ref, lr_ref, cbw_ref, out_ref):
    e = e_ref[...]
    lc = lc_ref[...]
    m = (lc == lr_ref[...]).astype(jnp.float32)
    s = jnp.dot(m, e, preferred_element_type=jnp.float32)
    cnt = jnp.sum(m, axis=1, keepdims=True)
    d = e.shape[1]
    parity = (lc % 2).astype(jnp.float32)
    cb = cbw_ref[:, :d] * (1.0 - parity) + cbw_ref[:, d:] * parity
    cbn = cb - _F * (cnt * cb - s)
    r = e - cbn
    out_ref[0, 0] = jnp.sum(r * r) / (e.shape[0] * e.shape[1])


def kernel(embedding, labels, centers, W, b):
    B, D = embedding.shape
    U = W.shape[1]

    NB = 2048
    et = embedding.T
    logits_t = pl.pallas_call(
        _matmul_t_body,
        grid=(pl.cdiv(U, NB),),
        in_specs=[
            pl.BlockSpec((D, NB), lambda i: (0, i)),
            pl.BlockSpec((D, B), lambda i: (0, 0)),
            pl.BlockSpec((NB,), lambda i: (i,)),
        ],
        out_specs=pl.BlockSpec((NB, B), lambda i: (i, 0)),
        out_shape=jax.ShapeDtypeStruct((U, B), jnp.float32),
        compiler_params=pltpu.CompilerParams(
            dimension_semantics=("parallel",)
        ),
    )(W, et, b)

    logits_t, labels_b = lax.optimization_barrier((logits_t, labels))
    centers2 = centers.reshape(U // 2, 2 * D)
    idx2 = labels_b // 2
    cbw = _make_sc_gather(B, U // 2, 2 * D)(centers2, idx2)

    loss = pl.pallas_call(
        _loss_body,
        out_specs=pl.BlockSpec(memory_space=pltpu.SMEM),
        out_shape=jax.ShapeDtypeStruct((1, 1), jnp.float32),
    )(embedding, labels.reshape(B, 1), labels.reshape(1, B), cbw)

    return (jnp.transpose(logits_t), loss[0, 0])


# speedup vs baseline: 2.9333x; 1.2700x over previous
import functools

import jax
import jax.numpy as jnp
from jax import lax
from jax.experimental import pallas as pl
from jax.experimental.pallas import tpu as pltpu
from jax.experimental.pallas import tpu_sc as plsc

ALPHA = 0.5
_F = 1.0 - ALPHA

_NC = 2
_NS = 16
_NW = _NC * _NS
_LANES = 16
_TW = 128


def _make_sc_gather_cols(n_idx, dim, n_cols, b0):
    per_w = n_idx // _NW
    nbuf = 4
    mesh = plsc.VectorSubcoreMesh(core_axis_name="c", subcore_axis_name="s")

    @functools.partial(
        pl.kernel,
        mesh=mesh,
        compiler_params=pltpu.CompilerParams(needs_layout_passes=False),
        out_type=jax.ShapeDtypeStruct((n_idx, _TW), jnp.float32),
        scratch_types=[
            pltpu.VMEM((per_w,), jnp.int32),
            pltpu.VMEM((per_w, _TW), jnp.float32),
            [pltpu.VMEM((dim, _TW), jnp.float32) for _ in range(nbuf)],
            pltpu.SemaphoreType.DMA,
        ],
    )
    def gather_cols(table_hbm, idx_hbm, out_hbm, idx_v, rows_v, tiles, sem):
        wid = lax.axis_index("s") * _NC + lax.axis_index("c")
        base = wid * per_w
        pltpu.sync_copy(idx_hbm.at[pl.ds(base, per_w)], idx_v)
        lane = lax.iota(jnp.int32, _LANES)
        for g in range(per_w // nbuf):
            v16 = idx_v[pl.ds((g * nbuf // _LANES) * _LANES, _LANES)]
            us, cps = [], []
            for j in range(nbuf):
                jj = (g * nbuf + j) % _LANES
                u = jnp.sum(jnp.where(lane == jj, v16, 0))
                ub = pl.multiple_of(
                    jnp.minimum((u // _TW) * _TW, b0), _TW
                )
                us.append(u - ub)
                cps.append(
                    pltpu.make_async_copy(
                        table_hbm.at[:, pl.ds(ub, _TW)], tiles[j], sem
                    )
                )
            for cp in cps:
                cp.start()
            for j in range(nbuf):
                cps[j].wait()
                c16 = jnp.broadcast_to(
                    jnp.minimum(us[j], _TW - 1), (_LANES,)
                )
                r = g * nbuf + j
                for t in range(dim // _LANES):
                    vals = plsc.load_gather(
                        tiles[j], [lane + t * _LANES, c16]
                    )
                    rows_v[r, pl.ds(t * _LANES, _LANES)] = vals
        pltpu.sync_copy(rows_v, out_hbm.at[pl.ds(base, per_w), :])

    return gather_cols


def _matmul_t_body(w_ref, et_ref, b_ref, out_ref):
    wt_et = lax.dot_general(
        w_ref[...], et_ref[...],
        (((0,), (0,)), ((), ())),
        preferred_element_type=jnp.float32,
    )
    ones = jnp.ones((1, out_ref.shape[1]), dtype=jnp.float32)
    bias = lax.dot_general(
        b_ref[...].reshape(1, b_ref.shape[0]), ones,
        (((0,), (0,)), ((), ())),
        preferred_element_type=jnp.float32,
    )
    out_ref[...] = wt_et + bias


def _make_loss_body(p0):
    def _loss_body(e_ref, lc_ref, lr_ref, cbw_ref, tail_ref, out_ref):
        e = e_ref[...]
        lc = lc_ref[...]
        m = (lc == lr_ref[...]).astype(jnp.float32)
        s = jnp.dot(m, e, preferred_element_type=jnp.float32)
        cnt = jnp.sum(m, axis=1, keepdims=True)
        d = e.shape[1]
        cb_sc = cbw_ref[:, :d]
        tw = tail_ref.shape[1]
        tcol = lax.broadcasted_iota(jnp.int32, (1, tw), 1) + p0
        onehot = (lc == tcol).astype(jnp.float32)
        cb_tail = lax.dot_general(
            onehot, tail_ref[...], (((1,), (1,)), ((), ())),
            preferred_element_type=jnp.float32,
        )
        cb = jnp.where(lc < p0, cb_sc, cb_tail)
        cbn = cb - _F * (cnt * cb - s)
        r = e - cbn
        out_ref[0, 0] = jnp.sum(r * r) / (e.shape[0] * e.shape[1])

    return _loss_body


def kernel(embedding, labels, centers, W, b):
    B, D = embedding.shape
    U = W.shape[1]

    NB = 2048
    et = embedding.T
    logits_t = pl.pallas_call(
        _matmul_t_body,
        grid=(pl.cdiv(U, NB),),
        in_specs=[
            pl.BlockSpec((D, NB), lambda i: (0, 0 + i)),
            pl.BlockSpec((D, B), lambda i: (0, 0)),
            pl.BlockSpec((NB,), lambda i: (i,)),
        ],
        out_specs=pl.BlockSpec((NB, B), lambda i: (i, 0)),
        out_shape=jax.ShapeDtypeStruct((U, B), jnp.float32),
        compiler_params=pltpu.CompilerParams(
            dimension_semantics=("parallel",)
        ),
    )(W, et, b)

    ct = centers.T
    b0 = ((U - _TW) // _TW) * _TW
    p0 = b0 + _TW
    cbw = _make_sc_gather_cols(B, D, U, b0)(ct, labels)

    tail = lax.slice(ct, (0, p0), (D, U))
    loss = pl.pallas_call(
        _make_loss_body(p0),
        out_specs=pl.BlockSpec(memory_space=pltpu.SMEM),
        out_shape=jax.ShapeDtypeStruct((1, 1), jnp.float32),
    )(embedding, labels.reshape(B, 1), labels.reshape(1, B), cbw, tail)

    return (jnp.transpose(logits_t), loss[0, 0])


# speedup vs baseline: 2.9774x; 1.0150x over previous
import functools

import jax
import jax.numpy as jnp
from jax import lax
from jax.experimental import pallas as pl
from jax.experimental.pallas import tpu as pltpu
from jax.experimental.pallas import tpu_sc as plsc

ALPHA = 0.5
_F = 1.0 - ALPHA

_NC = 2
_NS = 16
_NW = _NC * _NS
_LANES = 16
_TW = 128


def _make_sc_gather_cols(n_idx, dim, n_cols, b0):
    per_w = n_idx // _NW
    nbuf = 4
    mesh = plsc.VectorSubcoreMesh(core_axis_name="c", subcore_axis_name="s")

    @functools.partial(
        pl.kernel,
        mesh=mesh,
        compiler_params=pltpu.CompilerParams(needs_layout_passes=False),
        out_type=jax.ShapeDtypeStruct((n_idx, _TW), jnp.float32),
        scratch_types=[
            pltpu.VMEM((per_w,), jnp.int32),
            pltpu.VMEM((per_w, _TW), jnp.float32),
            [pltpu.VMEM((dim, _TW), jnp.float32) for _ in range(nbuf)],
            pltpu.SemaphoreType.DMA,
        ],
    )
    def gather_cols(table_hbm, idx_hbm, out_hbm, idx_v, rows_v, tiles, sem):
        wid = lax.axis_index("s") * _NC + lax.axis_index("c")
        base = wid * per_w
        pltpu.sync_copy(idx_hbm.at[pl.ds(base, per_w)], idx_v)
        lane = lax.iota(jnp.int32, _LANES)
        for g in range(per_w // nbuf):
            v16 = idx_v[pl.ds((g * nbuf // _LANES) * _LANES, _LANES)]
            us, cps = [], []
            for j in range(nbuf):
                jj = (g * nbuf + j) % _LANES
                u = jnp.sum(jnp.where(lane == jj, v16, 0))
                ub = pl.multiple_of(
                    jnp.minimum((u // _TW) * _TW, b0), _TW
                )
                us.append(u - ub)
                cps.append(
                    pltpu.make_async_copy(
                        table_hbm.at[:, pl.ds(ub, _TW)], tiles[j], sem
                    )
                )
            for cp in cps:
                cp.start()
            for j in range(nbuf):
                cps[j].wait()
                c16 = jnp.broadcast_to(
                    jnp.minimum(us[j], _TW - 1), (_LANES,)
                )
                r = g * nbuf + j
                for t in range(dim // _LANES):
                    vals = plsc.load_gather(
                        tiles[j], [lane + t * _LANES, c16]
                    )
                    rows_v[r, pl.ds(t * _LANES, _LANES)] = vals
        pltpu.sync_copy(rows_v, out_hbm.at[pl.ds(base, per_w), :])

    return gather_cols


def _matmul_t_body(w_ref, et_ref, b_ref, out_ref):
    wt_et = lax.dot_general(
        w_ref[...], et_ref[...],
        (((0,), (0,)), ((), ())),
        preferred_element_type=jnp.float32,
    )
    ones = jnp.ones((1, out_ref.shape[1]), dtype=jnp.float32)
    bias = lax.dot_general(
        b_ref[...].reshape(1, b_ref.shape[0]), ones,
        (((0,), (0,)), ((), ())),
        preferred_element_type=jnp.float32,
    )
    out_ref[...] = wt_et + bias


def _make_loss_body(p0):
    def _loss_body(e_ref, lc_ref, lr_ref, cbw_ref, tail_ref, out_ref):
        e = e_ref[...]
        lc = lc_ref[...]
        m = (lc == lr_ref[...]).astype(jnp.float32)
        s = jnp.dot(m, e, preferred_element_type=jnp.float32)
        cnt = jnp.sum(m, axis=1, keepdims=True)
        d = e.shape[1]
        cb_sc = cbw_ref[:, :d]
        tw = tail_ref.shape[1]
        tcol = lax.broadcasted_iota(jnp.int32, (1, tw), 1) + p0
        onehot = (lc == tcol).astype(jnp.float32)
        cb_tail = lax.dot_general(
            onehot, tail_ref[...], (((1,), (1,)), ((), ())),
            preferred_element_type=jnp.float32,
        )
        cb = jnp.where(lc < p0, cb_sc, cb_tail)
        cbn = cb - _F * (cnt * cb - s)
        r = e - cbn
        out_ref[0, 0] = jnp.sum(r * r) / (e.shape[0] * e.shape[1])

    return _loss_body


def kernel(embedding, labels, centers, W, b):
    B, D = embedding.shape
    U = W.shape[1]

    NB = 4096
    et = embedding.T
    logits_t = pl.pallas_call(
        _matmul_t_body,
        grid=(pl.cdiv(U, NB),),
        in_specs=[
            pl.BlockSpec((D, NB), lambda i: (0, 0 + i)),
            pl.BlockSpec((D, B), lambda i: (0, 0)),
            pl.BlockSpec((NB,), lambda i: (i,)),
        ],
        out_specs=pl.BlockSpec((NB, B), lambda i: (i, 0)),
        out_shape=jax.ShapeDtypeStruct((U, B), jnp.float32),
        compiler_params=pltpu.CompilerParams(
            dimension_semantics=("parallel",)
        ),
    )(W, et, b)

    ct = centers.T
    b0 = ((U - _TW) // _TW) * _TW
    p0 = b0 + _TW
    cbw = _make_sc_gather_cols(B, D, U, b0)(ct, labels)

    tail = lax.slice(ct, (0, p0), (D, U))
    loss = pl.pallas_call(
        _make_loss_body(p0),
        out_specs=pl.BlockSpec(memory_space=pltpu.SMEM),
        out_shape=jax.ShapeDtypeStruct((1, 1), jnp.float32),
    )(embedding, labels.reshape(B, 1), labels.reshape(1, B), cbw, tail)

    return (jnp.transpose(logits_t), loss[0, 0])


# speedup vs baseline: 2.9858x; 1.0028x over previous
import functools

import jax
import jax.numpy as jnp
from jax import lax
from jax.experimental import pallas as pl
from jax.experimental.pallas import tpu as pltpu
from jax.experimental.pallas import tpu_sc as plsc

ALPHA = 0.5
_F = 1.0 - ALPHA

_NC = 2
_NS = 16
_NW = _NC * _NS
_LANES = 16
_TW = 128


def _make_sc_gather_cols(n_idx, dim, n_cols, b0):
    per_w = n_idx // _NW
    nbuf = 4
    mesh = plsc.VectorSubcoreMesh(core_axis_name="c", subcore_axis_name="s")

    @functools.partial(
        pl.kernel,
        mesh=mesh,
        compiler_params=pltpu.CompilerParams(needs_layout_passes=False),
        out_type=jax.ShapeDtypeStruct((n_idx, _TW), jnp.float32),
        scratch_types=[
            pltpu.VMEM((per_w,), jnp.int32),
            pltpu.VMEM((per_w, _TW), jnp.float32),
            [pltpu.VMEM((dim, _TW), jnp.float32) for _ in range(nbuf)],
            pltpu.SemaphoreType.DMA,
        ],
    )
    def gather_cols(table_hbm, idx_hbm, out_hbm, idx_v, rows_v, tiles, sem):
        wid = lax.axis_index("s") * _NC + lax.axis_index("c")
        base = wid * per_w
        pltpu.sync_copy(idx_hbm.at[pl.ds(base, per_w)], idx_v)
        lane = lax.iota(jnp.int32, _LANES)
        for g in range(per_w // nbuf):
            v16 = idx_v[pl.ds((g * nbuf // _LANES) * _LANES, _LANES)]
            us, cps = [], []
            for j in range(nbuf):
                jj = (g * nbuf + j) % _LANES
                u = jnp.sum(jnp.where(lane == jj, v16, 0))
                ub = pl.multiple_of(
                    jnp.minimum((u // _TW) * _TW, b0), _TW
                )
                us.append(u - ub)
                cps.append(
                    pltpu.make_async_copy(
                        table_hbm.at[:, pl.ds(ub, _TW)], tiles[j], sem
                    )
                )
            for cp in cps:
                cp.start()
            for j in range(nbuf):
                cps[j].wait()
                c16 = jnp.broadcast_to(
                    jnp.minimum(us[j], _TW - 1), (_LANES,)
                )
                r = g * nbuf + j
                for t in range(dim // _LANES):
                    vals = plsc.load_gather(
                        tiles[j], [lane + t * _LANES, c16]
                    )
                    rows_v[r, pl.ds(t * _LANES, _LANES)] = vals
        pltpu.sync_copy(rows_v, out_hbm.at[pl.ds(base, per_w), :])

    return gather_cols


def _matmul_t_body(w_ref, et_ref, b_ref, out_ref):
    wt_et = lax.dot_general(
        w_ref[...], et_ref[...],
        (((0,), (0,)), ((), ())),
        preferred_element_type=jnp.float32,
    )
    ones = jnp.ones((1, out_ref.shape[1]), dtype=jnp.float32)
    bias = lax.dot_general(
        b_ref[...].reshape(1, b_ref.shape[0]), ones,
        (((0,), (0,)), ((), ())),
        preferred_element_type=jnp.float32,
    )
    out_ref[...] = wt_et + bias


def _make_loss_body(p0):
    def _loss_body(e_ref, lc_ref, lr_ref, cbw_ref, tail_ref, out_ref):
        e = e_ref[...]
        lc = lc_ref[...]
        m = (lc == lr_ref[...]).astype(jnp.float32)
        s = jnp.dot(m, e, preferred_element_type=jnp.float32)
        cnt = jnp.sum(m, axis=1, keepdims=True)
        d = e.shape[1]
        cb_sc = cbw_ref[:, :d]
        tw = tail_ref.shape[1]
        tcol = lax.broadcasted_iota(jnp.int32, (1, tw), 1) + p0
        onehot = (lc == tcol).astype(jnp.float32)
        cb_tail = lax.dot_general(
            onehot, tail_ref[...], (((1,), (1,)), ((), ())),
            preferred_element_type=jnp.float32,
        )
        cb = jnp.where(lc < p0, cb_sc, cb_tail)
        cbn = cb - _F * (cnt * cb - s)
        r = e - cbn
        out_ref[0, 0] = jnp.sum(r * r) / (e.shape[0] * e.shape[1])

    return _loss_body


def kernel(embedding, labels, centers, W, b):
    B, D = embedding.shape
    U = W.shape[1]

    NB = 5120
    et = embedding.T
    logits_t = pl.pallas_call(
        _matmul_t_body,
        grid=(pl.cdiv(U, NB),),
        in_specs=[
            pl.BlockSpec((D, NB), lambda i: (0, 0 + i)),
            pl.BlockSpec((D, B), lambda i: (0, 0)),
            pl.BlockSpec((NB,), lambda i: (i,)),
        ],
        out_specs=pl.BlockSpec((NB, B), lambda i: (i, 0)),
        out_shape=jax.ShapeDtypeStruct((U, B), jnp.float32),
        compiler_params=pltpu.CompilerParams(
            dimension_semantics=("parallel",)
        ),
    )(W, et, b)

    ct = centers.T
    b0 = ((U - _TW) // _TW) * _TW
    p0 = b0 + _TW
    cbw = _make_sc_gather_cols(B, D, U, b0)(ct, labels)

    tail = lax.slice(ct, (0, p0), (D, U))
    loss = pl.pallas_call(
        _make_loss_body(p0),
        out_specs=pl.BlockSpec(memory_space=pltpu.SMEM),
        out_shape=jax.ShapeDtypeStruct((1, 1), jnp.float32),
    )(embedding, labels.reshape(B, 1), labels.reshape(1, B), cbw, tail)

    return (jnp.transpose(logits_t), loss[0, 0])


# speedup vs baseline: 2.9881x; 1.0008x over previous
import functools

import jax
import jax.numpy as jnp
from jax import lax
from jax.experimental import pallas as pl
from jax.experimental.pallas import tpu as pltpu
from jax.experimental.pallas import tpu_sc as plsc

ALPHA = 0.5
_F = 1.0 - ALPHA

_NC = 2
_NS = 16
_NW = _NC * _NS
_LANES = 16
_TW = 128


def _make_sc_gather_cols(n_idx, dim, n_cols, b0):
    per_w = n_idx // _NW
    nbuf = 4
    mesh = plsc.VectorSubcoreMesh(core_axis_name="c", subcore_axis_name="s")

    @functools.partial(
        pl.kernel,
        mesh=mesh,
        compiler_params=pltpu.CompilerParams(needs_layout_passes=False),
        out_type=jax.ShapeDtypeStruct((n_idx, _TW), jnp.float32),
        scratch_types=[
            pltpu.VMEM((per_w,), jnp.int32),
            pltpu.VMEM((per_w, _TW), jnp.float32),
            [pltpu.VMEM((dim, _TW), jnp.float32) for _ in range(nbuf)],
            pltpu.SemaphoreType.DMA,
        ],
    )
    def gather_cols(table_hbm, idx_hbm, out_hbm, idx_v, rows_v, tiles, sem):
        wid = lax.axis_index("s") * _NC + lax.axis_index("c")
        base = wid * per_w
        pltpu.sync_copy(idx_hbm.at[pl.ds(base, per_w)], idx_v)
        lane = lax.iota(jnp.int32, _LANES)
        for g in range(per_w // nbuf):
            v16 = idx_v[pl.ds((g * nbuf // _LANES) * _LANES, _LANES)]
            us, cps = [], []
            for j in range(nbuf):
                jj = (g * nbuf + j) % _LANES
                u = jnp.sum(jnp.where(lane == jj, v16, 0))
                ub = pl.multiple_of(
                    jnp.minimum((u // _TW) * _TW, b0), _TW
                )
                us.append(u - ub)
                cps.append(
                    pltpu.make_async_copy(
                        table_hbm.at[:, pl.ds(ub, _TW)], tiles[j], sem
                    )
                )
            for cp in cps:
                cp.start()
            for j in range(nbuf):
                cps[j].wait()
                c16 = jnp.broadcast_to(
                    jnp.minimum(us[j], _TW - 1), (_LANES,)
                )
                r = g * nbuf + j
                for t in range(dim // _LANES):
                    vals = plsc.load_gather(
                        tiles[j], [lane + t * _LANES, c16]
                    )
                    rows_v[r, pl.ds(t * _LANES, _LANES)] = vals
        pltpu.sync_copy(rows_v, out_hbm.at[pl.ds(base, per_w), :])

    return gather_cols


def _matmul_t_body(w_ref, et_ref, b_ref, out_ref):
    wt_et = lax.dot_general(
        w_ref[...], et_ref[...],
        (((0,), (0,)), ((), ())),
        preferred_element_type=jnp.float32,
    )
    ones = jnp.ones((1, out_ref.shape[1]), dtype=jnp.float32)
    bias = lax.dot_general(
        b_ref[...].reshape(1, b_ref.shape[0]), ones,
        (((0,), (0,)), ((), ())),
        preferred_element_type=jnp.float32,
    )
    out_ref[...] = wt_et + bias


def _make_loss_body(p0):
    def _loss_body(e_ref, lc_ref, lr_ref, cbw_ref, tail_ref, out_ref):
        e = e_ref[...]
        lc = lc_ref[...]
        m = (lc == lr_ref[...]).astype(jnp.float32)
        s = jnp.dot(m, e, preferred_element_type=jnp.float32)
        cnt = jnp.sum(m, axis=1, keepdims=True)
        d = e.shape[1]
        cb_sc = cbw_ref[:, :d]
        tw = tail_ref.shape[1]
        tcol = lax.broadcasted_iota(jnp.int32, (1, tw), 1) + p0
        onehot = (lc == tcol).astype(jnp.float32)
        cb_tail = lax.dot_general(
            onehot, tail_ref[...], (((1,), (1,)), ((), ())),
            preferred_element_type=jnp.float32,
        )
        cb = jnp.where(lc < p0, cb_sc, cb_tail)
        cbn = cb - _F * (cnt * cb - s)
        r = e - cbn
        out_ref[0, 0] = jnp.sum(r * r) / (e.shape[0] * e.shape[1])

    return _loss_body


def kernel(embedding, labels, centers, W, b):
    B, D = embedding.shape
    U = W.shape[1]

    NB = 5120
    et = embedding.T
    logits_t = pl.pallas_call(
        _matmul_t_body,
        grid=(pl.cdiv(U, NB),),
        in_specs=[
            pl.BlockSpec((D, NB), lambda i: (0, i)),
            pl.BlockSpec((D, B), lambda i: (0, 0)),
            pl.BlockSpec((NB,), lambda i: (i,)),
        ],
        out_specs=pl.BlockSpec((NB, B), lambda i: (i, 0)),
        out_shape=jax.ShapeDtypeStruct((U, B), jnp.float32),
        compiler_params=pltpu.CompilerParams(
            dimension_semantics=("parallel",)
        ),
    )(W, et, b)

    ct = centers.T
    b0 = ((U - _TW) // _TW) * _TW
    p0 = b0 + _TW
    cbw = _make_sc_gather_cols(B, D, U, b0)(ct, labels)

    tail = lax.slice(ct, (0, p0), (D, U))
    loss = pl.pallas_call(
        _make_loss_body(p0),
        out_specs=pl.BlockSpec(memory_space=pltpu.SMEM),
        out_shape=jax.ShapeDtypeStruct((1, 1), jnp.float32),
    )(embedding, labels.reshape(B, 1), labels.reshape(1, B), cbw, tail)

    return (jnp.transpose(logits_t), loss[0, 0])
